# Initial kernel scaffold; baseline (speedup 1.0000x reference)
import jax
import jax.numpy as jnp
from jax.experimental import pallas as pl


def kernel(inputs, mapping):
    raise NotImplementedError("write your pallas kernel here")


# speedup vs baseline: 217.4077x; 217.4077x over previous
import dataclasses
import functools

import jax
import jax.numpy as jnp
from jax import lax
from jax.experimental import pallas as pl
from jax.experimental.pallas import tpu as pltpu
from jax.experimental.pallas import tpu_sc as plsc

_NC = 2
_NS = 16
_NW = _NC * _NS
_L = 16

_TOTAL = 16384 * 200
_PER_W = _TOTAL // _NW
_BS = 12800
_NBLK = _PER_W // _BS


def _make_sc_kernel():
    mesh = plsc.VectorSubcoreMesh(core_axis_name="c", subcore_axis_name="s")

    cp = pltpu.CompilerParams()
    if "needs_layout_passes" in pltpu.CompilerParams.__dataclass_fields__:
        cp = dataclasses.replace(cp, needs_layout_passes=False)

    @functools.partial(
        pl.kernel,
        mesh=mesh,
        out_type=jax.ShapeDtypeStruct((_TOTAL,), jnp.int32),
        scratch_types=[
            pltpu.VMEM((128,), jnp.int32),
            pltpu.VMEM((_BS,), jnp.int32),
            pltpu.VMEM((_BS,), jnp.int32),
        ],
        compiler_params=cp,
    )
    def sc_kernel(in_hbm, map_hbm, out_hbm, table_v, in_v, out_v):
        wid = lax.axis_index("s") * _NC + lax.axis_index("c")
        base = wid * _PER_W

        pltpu.sync_copy(map_hbm, table_v)

        def do_block(blk, _):
            off = base + blk * _BS
            pltpu.sync_copy(in_hbm.at[pl.ds(off, _BS)], in_v)

            def translate(i, _):
                codes = in_v[pl.ds(i * _L, _L)]
                out_v[pl.ds(i * _L, _L)] = plsc.load_gather(table_v, [codes])
                return 0

            lax.fori_loop(0, _BS // _L, translate, 0)
            pltpu.sync_copy(out_v, out_hbm.at[pl.ds(off, _BS)])
            return 0

        lax.fori_loop(0, _NBLK, do_block, 0)

    return sc_kernel


_sc_kernel = _make_sc_kernel()


@jax.jit
def kernel(inputs, mapping):
    flat = inputs.reshape(_TOTAL)
    out = _sc_kernel(flat, mapping)
    return out.reshape(inputs.shape)


# speedup vs baseline: 256.4863x; 1.1797x over previous
import dataclasses
import functools

import jax
import jax.numpy as jnp
from jax import lax
from jax.experimental import pallas as pl
from jax.experimental.pallas import tpu as pltpu
from jax.experimental.pallas import tpu_sc as plsc

_NC = 2
_NS = 16
_NW = _NC * _NS
_L = 16

_TOTAL = 16384 * 200
_PER_W = _TOTAL // _NW
_BS = 25600
_NBLK = _PER_W // _BS
_UNROLL = 8


def _make_sc_kernel():
    mesh = plsc.VectorSubcoreMesh(core_axis_name="c", subcore_axis_name="s")

    cp = pltpu.CompilerParams()
    if "needs_layout_passes" in pltpu.CompilerParams.__dataclass_fields__:
        cp = dataclasses.replace(cp, needs_layout_passes=False)

    @functools.partial(
        pl.kernel,
        mesh=mesh,
        out_type=jax.ShapeDtypeStruct((_TOTAL,), jnp.int32),
        scratch_types=[
            pltpu.VMEM((128,), jnp.int32),
            pltpu.VMEM((_BS,), jnp.int32),
            pltpu.VMEM((_BS,), jnp.int32),
            pltpu.VMEM((_BS,), jnp.int32),
            pltpu.VMEM((_BS,), jnp.int32),
            pltpu.SemaphoreType.DMA,
            pltpu.SemaphoreType.DMA,
            pltpu.SemaphoreType.DMA,
            pltpu.SemaphoreType.DMA,
        ],
        compiler_params=cp,
    )
    def sc_kernel(in_hbm, map_hbm, out_hbm,
                  table_v, in_v0, in_v1, out_v0, out_v1,
                  sin0, sin1, sout0, sout1):
        wid = lax.axis_index("s") * _NC + lax.axis_index("c")
        base = wid * _PER_W

        in_bufs = (in_v0, in_v1)
        out_bufs = (out_v0, out_v1)
        sins = (sin0, sin1)
        souts = (sout0, sout1)

        pltpu.sync_copy(map_hbm, table_v)

        def in_copy(blk):
            b = blk % 2
            return pltpu.make_async_copy(
                in_hbm.at[pl.ds(base + blk * _BS, _BS)], in_bufs[b], sins[b])

        def out_copy(blk):
            b = blk % 2
            return pltpu.make_async_copy(
                out_bufs[b], out_hbm.at[pl.ds(base + blk * _BS, _BS)], souts[b])

        in_copy(0).start()
        for blk in range(_NBLK):
            b = blk % 2
            in_copy(blk).wait()
            if blk + 1 < _NBLK:
                in_copy(blk + 1).start()
            if blk >= 2:
                out_copy(blk - 2).wait()

            in_v = in_bufs[b]
            out_v = out_bufs[b]

            def translate(j, _, in_v=in_v, out_v=out_v):
                off = j * (_L * _UNROLL)
                for u in range(_UNROLL):
                    s = pl.ds(off + u * _L, _L)
                    out_v[s] = plsc.load_gather(table_v, [in_v[s]])
                return 0

            lax.fori_loop(0, _BS // (_L * _UNROLL), translate, 0)
            out_copy(blk).start()

        for blk in range(max(_NBLK - 2, 0), _NBLK):
            out_copy(blk).wait()

    return sc_kernel


_sc_kernel = _make_sc_kernel()


@jax.jit
def kernel(inputs, mapping):
    flat = inputs.reshape(_TOTAL)
    out = _sc_kernel(flat, mapping)
    return out.reshape(inputs.shape)


# speedup vs baseline: 421.0200x; 1.6415x over previous
import dataclasses
import functools

import jax
import jax.numpy as jnp
from jax import lax
from jax.experimental import pallas as pl
from jax.experimental.pallas import tpu as pltpu
from jax.experimental.pallas import tpu_sc as plsc

_NC = 2
_NS = 16
_NW = _NC * _NS
_L = 16

_ROWS = 16384
_COLS = 200
_ROWS_W = _ROWS // _NW
_BR = 64
_NBLK = _ROWS_W // _BR
_GROUP_STARTS = tuple(range(0, _COLS - _L, _L)) + (_COLS - _L,)


def _make_sc_kernel():
    mesh = plsc.VectorSubcoreMesh(core_axis_name="c", subcore_axis_name="s")

    cp = pltpu.CompilerParams()
    if "needs_layout_passes" in pltpu.CompilerParams.__dataclass_fields__:
        cp = dataclasses.replace(cp, needs_layout_passes=False)

    @functools.partial(
        pl.kernel,
        mesh=mesh,
        out_type=jax.ShapeDtypeStruct((_ROWS, _COLS), jnp.int32),
        scratch_types=[
            pltpu.VMEM((128,), jnp.int32),
            pltpu.VMEM((_BR, _COLS), jnp.int32),
            pltpu.VMEM((_BR, _COLS), jnp.int32),
            pltpu.VMEM((_BR, _COLS), jnp.int32),
            pltpu.VMEM((_BR, _COLS), jnp.int32),
            pltpu.SemaphoreType.DMA,
            pltpu.SemaphoreType.DMA,
            pltpu.SemaphoreType.DMA,
            pltpu.SemaphoreType.DMA,
        ],
        compiler_params=cp,
    )
    def sc_kernel(in_hbm, map_hbm, out_hbm,
                  table_v, in_v0, in_v1, out_v0, out_v1,
                  sin0, sin1, sout0, sout1):
        wid = lax.axis_index("s") * _NC + lax.axis_index("c")
        base = wid * _ROWS_W

        in_bufs = (in_v0, in_v1)
        out_bufs = (out_v0, out_v1)
        sins = (sin0, sin1)
        souts = (sout0, sout1)

        pltpu.sync_copy(map_hbm, table_v)

        def in_copy(blk):
            b = blk % 2
            return pltpu.make_async_copy(
                in_hbm.at[pl.ds(base + blk * _BR, _BR), :], in_bufs[b], sins[b])

        def out_copy(blk):
            b = blk % 2
            return pltpu.make_async_copy(
                out_bufs[b], out_hbm.at[pl.ds(base + blk * _BR, _BR), :], souts[b])

        in_copy(0).start()
        for blk in range(_NBLK):
            b = blk % 2
            in_copy(blk).wait()
            if blk + 1 < _NBLK:
                in_copy(blk + 1).start()
            if blk >= 2:
                out_copy(blk - 2).wait()

            in_v = in_bufs[b]
            out_v = out_bufs[b]

            def translate(r, _, in_v=in_v, out_v=out_v):
                for c0 in _GROUP_STARTS:
                    s = pl.ds(c0, _L)
                    out_v[r, s] = plsc.load_gather(table_v, [in_v[r, s]])
                return 0

            lax.fori_loop(0, _BR, translate, 0)
            out_copy(blk).start()

        for blk in range(max(_NBLK - 2, 0), _NBLK):
            out_copy(blk).wait()

    return sc_kernel


_sc_kernel = _make_sc_kernel()


@jax.jit
def kernel(inputs, mapping):
    return _sc_kernel(inputs, mapping)


# speedup vs baseline: 650.3259x; 1.5446x over previous
import dataclasses
import functools

import jax
import jax.numpy as jnp
from jax import lax
from jax.experimental import pallas as pl
from jax.experimental.pallas import tpu as pltpu
from jax.experimental.pallas import tpu_sc as plsc

_NC = 2
_NS = 16
_NW = _NC * _NS
_L = 16

_R = 200
_C = 16384
_C_W = _C // _NW
_BC = 128
_NBLK = _C_W // _BC


def _make_sc_kernel():
    mesh = plsc.VectorSubcoreMesh(core_axis_name="c", subcore_axis_name="s")

    cp = pltpu.CompilerParams()
    if "needs_layout_passes" in pltpu.CompilerParams.__dataclass_fields__:
        cp = dataclasses.replace(cp, needs_layout_passes=False)

    @functools.partial(
        pl.kernel,
        mesh=mesh,
        out_type=jax.ShapeDtypeStruct((_R, _C), jnp.int32),
        scratch_types=[
            pltpu.VMEM((128,), jnp.int32),
            pltpu.VMEM((_R, _BC), jnp.int32),
            pltpu.VMEM((_R, _BC), jnp.int32),
            pltpu.VMEM((_R, _BC), jnp.int32),
            pltpu.VMEM((_R, _BC), jnp.int32),
            pltpu.SemaphoreType.DMA,
            pltpu.SemaphoreType.DMA,
            pltpu.SemaphoreType.DMA,
            pltpu.SemaphoreType.DMA,
        ],
        compiler_params=cp,
    )
    def sc_kernel(in_hbm, map_hbm, out_hbm,
                  table_v, in_v0, in_v1, out_v0, out_v1,
                  sin0, sin1, sout0, sout1):
        wid = lax.axis_index("s") * _NC + lax.axis_index("c")
        base = wid * _C_W

        in_bufs = (in_v0, in_v1)
        out_bufs = (out_v0, out_v1)
        sins = (sin0, sin1)
        souts = (sout0, sout1)

        pltpu.sync_copy(map_hbm, table_v)

        def in_copy(blk):
            b = blk % 2
            return pltpu.make_async_copy(
                in_hbm.at[:, pl.ds(base + blk * _BC, _BC)], in_bufs[b], sins[b])

        def out_copy(blk):
            b = blk % 2
            return pltpu.make_async_copy(
                out_bufs[b], out_hbm.at[:, pl.ds(base + blk * _BC, _BC)], souts[b])

        in_copy(0).start()
        for blk in range(_NBLK):
            b = blk % 2
            in_copy(blk).wait()
            if blk + 1 < _NBLK:
                in_copy(blk + 1).start()
            if blk >= 2:
                out_copy(blk - 2).wait()

            in_v = in_bufs[b]
            out_v = out_bufs[b]

            def translate(r, _, in_v=in_v, out_v=out_v):
                for g in range(_BC // _L):
                    s = pl.ds(g * _L, _L)
                    out_v[r, s] = plsc.load_gather(table_v, [in_v[r, s]])
                return 0

            lax.fori_loop(0, _R, translate, 0)
            out_copy(blk).start()

        for blk in range(max(_NBLK - 2, 0), _NBLK):
            out_copy(blk).wait()

    return sc_kernel


_sc_kernel = _make_sc_kernel()


@jax.jit
def kernel(inputs, mapping):
    out_t = _sc_kernel(inputs.T, mapping)
    return out_t.T


# speedup vs baseline: 965.6013x; 1.4848x over previous
import dataclasses
import functools

import jax
import jax.numpy as jnp
from jax import lax
from jax.experimental import pallas as pl
from jax.experimental.pallas import tpu as pltpu
from jax.experimental.pallas import tpu_sc as plsc

_NC = 2
_NS = 16
_NW = _NC * _NS
_L = 16

_R = 200
_C = 16384
_C_W = _C // _NW
_BC = 128
_NBLK = _C_W // _BC


def _make_sc_kernel():
    mesh = plsc.VectorSubcoreMesh(core_axis_name="c", subcore_axis_name="s")

    cp = pltpu.CompilerParams()
    if "needs_layout_passes" in pltpu.CompilerParams.__dataclass_fields__:
        cp = dataclasses.replace(cp, needs_layout_passes=False)

    @functools.partial(
        pl.kernel,
        mesh=mesh,
        out_type=jax.ShapeDtypeStruct((_R, _C), jnp.int32),
        scratch_types=[
            pltpu.VMEM((128,), jnp.int32),
            pltpu.VMEM((_R, _BC), jnp.int32),
            pltpu.VMEM((_R, _BC), jnp.int32),
            pltpu.VMEM((_R, _BC), jnp.int32),
            pltpu.VMEM((_R, _BC), jnp.int32),
            pltpu.SemaphoreType.DMA,
            pltpu.SemaphoreType.DMA,
            pltpu.SemaphoreType.DMA,
            pltpu.SemaphoreType.DMA,
        ],
        compiler_params=cp,
    )
    def sc_kernel(in_hbm, map_hbm, out_hbm,
                  table_v, in_v0, in_v1, out_v0, out_v1,
                  sin0, sin1, sout0, sout1):
        wid = lax.axis_index("s") * _NC + lax.axis_index("c")
        base = wid * _C_W

        in_bufs = (in_v0, in_v1)
        out_bufs = (out_v0, out_v1)
        sins = (sin0, sin1)
        souts = (sout0, sout1)

        pltpu.sync_copy(map_hbm, table_v)

        def in_copy(blk):
            b = blk % 2
            return pltpu.make_async_copy(
                in_hbm.at[:, pl.ds(base + blk * _BC, _BC)], in_bufs[b], sins[b])

        def out_copy(blk):
            b = blk % 2
            return pltpu.make_async_copy(
                out_bufs[b], out_hbm.at[:, pl.ds(base + blk * _BC, _BC)], souts[b])

        in_copy(0).start()
        for blk in range(_NBLK):
            b = blk % 2
            in_copy(blk).wait()
            if blk + 1 < _NBLK:
                in_copy(blk + 1).start()
            if blk >= 2:
                out_copy(blk - 2).wait()

            in_v = in_bufs[b]
            out_v = out_bufs[b]

            @plsc.parallel_loop(0, _R, step=1, unroll=4)
            def translate(r, in_v=in_v, out_v=out_v):
                for g in range(_BC // _L):
                    s = pl.ds(g * _L, _L)
                    out_v[r, s] = plsc.load_gather(table_v, [in_v[r, s]])
            out_copy(blk).start()

        for blk in range(max(_NBLK - 2, 0), _NBLK):
            out_copy(blk).wait()

    return sc_kernel


_sc_kernel = _make_sc_kernel()


@jax.jit
def kernel(inputs, mapping):
    out_t = _sc_kernel(inputs.T, mapping)
    return out_t.T


# speedup vs baseline: 991.4999x; 1.0268x over previous
import dataclasses
import functools

import jax
import jax.numpy as jnp
from jax import lax
from jax.experimental import pallas as pl
from jax.experimental.pallas import tpu as pltpu
from jax.experimental.pallas import tpu_sc as plsc

_NC = 2
_NS = 16
_NW = _NC * _NS
_L = 16

_R = 200
_C = 16384
_C_W = _C // _NW
_BC = 128
_NBLK = _C_W // _BC


def _make_sc_kernel():
    mesh = plsc.VectorSubcoreMesh(core_axis_name="c", subcore_axis_name="s")

    cp = pltpu.CompilerParams()
    if "needs_layout_passes" in pltpu.CompilerParams.__dataclass_fields__:
        cp = dataclasses.replace(cp, needs_layout_passes=False)

    @functools.partial(
        pl.kernel,
        mesh=mesh,
        out_type=jax.ShapeDtypeStruct((_R, _C), jnp.int32),
        scratch_types=[
            pltpu.VMEM((128,), jnp.int32),
            pltpu.VMEM((_R, _BC), jnp.int32),
            pltpu.VMEM((_R, _BC), jnp.int32),
            pltpu.VMEM((_R, _BC), jnp.int32),
            pltpu.VMEM((_R, _BC), jnp.int32),
            pltpu.SemaphoreType.DMA,
            pltpu.SemaphoreType.DMA,
            pltpu.SemaphoreType.DMA,
            pltpu.SemaphoreType.DMA,
        ],
        compiler_params=cp,
    )
    def sc_kernel(in_hbm, map_hbm, out_hbm,
                  table_v, in_v0, in_v1, out_v0, out_v1,
                  sin0, sin1, sout0, sout1):
        wid = lax.axis_index("s") * _NC + lax.axis_index("c")
        base = wid * _C_W

        in_bufs = (in_v0, in_v1)
        out_bufs = (out_v0, out_v1)
        sins = (sin0, sin1)
        souts = (sout0, sout1)

        pltpu.sync_copy(map_hbm, table_v)

        def in_copy(blk):
            b = blk % 2
            return pltpu.make_async_copy(
                in_hbm.at[:, pl.ds(base + blk * _BC, _BC)], in_bufs[b], sins[b])

        def out_copy(blk):
            b = blk % 2
            return pltpu.make_async_copy(
                out_bufs[b], out_hbm.at[:, pl.ds(base + blk * _BC, _BC)], souts[b])

        def in_copy_b(blk, b):
            return pltpu.make_async_copy(
                in_hbm.at[:, pl.ds(base + blk * _BC, _BC)], in_bufs[b], sins[b])

        def out_copy_b(blk, b):
            return pltpu.make_async_copy(
                out_bufs[b], out_hbm.at[:, pl.ds(base + blk * _BC, _BC)], souts[b])

        in_copy(0).start()

        def do_pair(p, _):
            blk0 = 2 * p
            for b in (0, 1):
                blk = blk0 + b
                in_copy_b(blk, b).wait()
                nxt = blk + 1

                @pl.when(nxt < _NBLK)
                def _():
                    in_copy_b(nxt, 1 - b).start()

                @pl.when(blk >= 2)
                def _():
                    out_copy_b(blk - 2, b).wait()

                in_v = in_bufs[b]
                out_v = out_bufs[b]

                @plsc.parallel_loop(0, _R, step=1, unroll=4)
                def translate(r, in_v=in_v, out_v=out_v):
                    for g in range(_BC // _L):
                        s = pl.ds(g * _L, _L)
                        out_v[r, s] = plsc.load_gather(table_v, [in_v[r, s]])

                out_copy_b(blk, b).start()
            return 0

        lax.fori_loop(0, _NBLK // 2, do_pair, 0)

        for blk in range(max(_NBLK - 2, 0), _NBLK):
            out_copy(blk).wait()

    return sc_kernel


_sc_kernel = _make_sc_kernel()


@jax.jit
def kernel(inputs, mapping):
    out_t = _sc_kernel(inputs.T, mapping)
    return out_t.T


# speedup vs baseline: 993.8282x; 1.0023x over previous
import dataclasses
import functools

import jax
import jax.numpy as jnp
from jax import lax
from jax.experimental import pallas as pl
from jax.experimental.pallas import tpu as pltpu
from jax.experimental.pallas import tpu_sc as plsc

_NC = 2
_NS = 16
_NW = _NC * _NS
_L = 16

_R = 200
_C = 16384
_C_W = _C // _NW
_BC = 128
_NBLK = _C_W // _BC


def _make_sc_kernel():
    mesh = plsc.VectorSubcoreMesh(core_axis_name="c", subcore_axis_name="s")

    cp = pltpu.CompilerParams()
    if "needs_layout_passes" in pltpu.CompilerParams.__dataclass_fields__:
        cp = dataclasses.replace(cp, needs_layout_passes=False)

    @functools.partial(
        pl.kernel,
        mesh=mesh,
        out_type=jax.ShapeDtypeStruct((_R, _C), jnp.int32),
        scratch_types=[
            pltpu.VMEM((128,), jnp.int32),
            pltpu.VMEM((_R, _BC), jnp.int32),
            pltpu.VMEM((_R, _BC), jnp.int32),
            pltpu.VMEM((_R, _BC), jnp.int32),
            pltpu.VMEM((_R, _BC), jnp.int32),
            pltpu.SemaphoreType.DMA,
            pltpu.SemaphoreType.DMA,
            pltpu.SemaphoreType.DMA,
            pltpu.SemaphoreType.DMA,
        ],
        compiler_params=cp,
    )
    def sc_kernel(in_hbm, map_hbm, out_hbm,
                  table_v, in_v0, in_v1, out_v0, out_v1,
                  sin0, sin1, sout0, sout1):
        wid = lax.axis_index("s") * _NC + lax.axis_index("c")
        base = wid * _C_W

        in_bufs = (in_v0, in_v1)
        out_bufs = (out_v0, out_v1)
        sins = (sin0, sin1)
        souts = (sout0, sout1)

        pltpu.sync_copy(map_hbm, table_v)

        def in_copy(blk):
            b = blk % 2
            return pltpu.make_async_copy(
                in_hbm.at[:, pl.ds(base + blk * _BC, _BC)], in_bufs[b], sins[b])

        def out_copy(blk):
            b = blk % 2
            return pltpu.make_async_copy(
                out_bufs[b], out_hbm.at[:, pl.ds(base + blk * _BC, _BC)], souts[b])

        def in_copy_b(blk, b):
            return pltpu.make_async_copy(
                in_hbm.at[:, pl.ds(base + blk * _BC, _BC)], in_bufs[b], sins[b])

        def out_copy_b(blk, b):
            return pltpu.make_async_copy(
                out_bufs[b], out_hbm.at[:, pl.ds(base + blk * _BC, _BC)], souts[b])

        in_copy(0).start()

        def do_pair(p, _):
            blk0 = 2 * p
            for b in (0, 1):
                blk = blk0 + b
                in_copy_b(blk, b).wait()
                nxt = blk + 1

                @pl.when(nxt < _NBLK)
                def _():
                    in_copy_b(nxt, 1 - b).start()

                @pl.when(blk >= 2)
                def _():
                    out_copy_b(blk - 2, b).wait()

                in_v = in_bufs[b]
                out_v = out_bufs[b]

                @plsc.parallel_loop(0, _R, step=1, unroll=2)
                def translate(r, in_v=in_v, out_v=out_v):
                    for g in range(_BC // _L):
                        s = pl.ds(g * _L, _L)
                        out_v[r, s] = plsc.load_gather(table_v, [in_v[r, s]])

                out_copy_b(blk, b).start()
            return 0

        lax.fori_loop(0, _NBLK // 2, do_pair, 0)

        for blk in range(max(_NBLK - 2, 0), _NBLK):
            out_copy(blk).wait()

    return sc_kernel


_sc_kernel = _make_sc_kernel()


@jax.jit
def kernel(inputs, mapping):
    out_t = _sc_kernel(inputs.T, mapping)
    return out_t.T


# speedup vs baseline: 1024.0192x; 1.0304x over previous
import dataclasses
import functools

import jax
import jax.numpy as jnp
from jax import lax
from jax.experimental import pallas as pl
from jax.experimental.pallas import tpu as pltpu
from jax.experimental.pallas import tpu_sc as plsc

_NC = 2
_NS = 16
_NW = _NC * _NS
_L = 16

_R = 200
_C = 16384
_C_W = _C // _NW
_BC = 128
_NBLK = _C_W // _BC


def _make_sc_kernel():
    mesh = plsc.VectorSubcoreMesh(core_axis_name="c", subcore_axis_name="s")

    cp = pltpu.CompilerParams()
    if "needs_layout_passes" in pltpu.CompilerParams.__dataclass_fields__:
        cp = dataclasses.replace(cp, needs_layout_passes=False)

    @functools.partial(
        pl.kernel,
        mesh=mesh,
        out_type=jax.ShapeDtypeStruct((_R, _C), jnp.int32),
        scratch_types=[
            pltpu.VMEM((128,), jnp.int32),
            pltpu.VMEM((2, _R, _BC), jnp.int32),
            pltpu.VMEM((2, _R, _BC), jnp.int32),
            pltpu.SemaphoreType.DMA((2,)),
            pltpu.SemaphoreType.DMA((2,)),
        ],
        compiler_params=cp,
    )
    def sc_kernel(in_hbm, map_hbm, out_hbm, table_v, in_b, out_b, sin, sout):
        wid = lax.axis_index("s") * _NC + lax.axis_index("c")
        base = wid * _C_W

        def in_copy(blk, par):
            return pltpu.make_async_copy(
                in_hbm.at[:, pl.ds(base + blk * _BC, _BC)],
                in_b.at[par], sin.at[par])

        def out_copy(blk, par):
            return pltpu.make_async_copy(
                out_b.at[par], out_hbm.at[:, pl.ds(base + blk * _BC, _BC)],
                sout.at[par])

        in_copy(0, 0).start()
        pltpu.sync_copy(map_hbm, table_v)

        def do_blk(blk, _):
            par = lax.rem(blk, 2)
            in_copy(blk, par).wait()

            @pl.when(blk + 1 < _NBLK)
            def _():
                in_copy(blk + 1, 1 - par).start()

            @pl.when(blk >= 2)
            def _():
                out_copy(blk - 2, par).wait()

            @plsc.parallel_loop(0, _R, step=1, unroll=4)
            def translate(r):
                for g in range(_BC // _L):
                    s = pl.ds(g * _L, _L)
                    out_b[par, r, s] = plsc.load_gather(
                        table_v, [in_b[par, r, s]])

            out_copy(blk, par).start()
            return 0

        lax.fori_loop(0, _NBLK, do_blk, 0)
        out_copy(_NBLK - 2, (_NBLK - 2) % 2).wait()
        out_copy(_NBLK - 1, (_NBLK - 1) % 2).wait()

    return sc_kernel


_sc_kernel = _make_sc_kernel()


@jax.jit
def kernel(inputs, mapping):
    out_t = _sc_kernel(inputs.T, mapping)
    return out_t.T
